# baseline (speedup 1.0000x reference)
import functools

import jax
import jax.numpy as jnp
from jax import lax
from jax.experimental import pallas as pl
from jax.experimental.pallas import tpu as pltpu
from jax.experimental.pallas import tpu_sc as plsc

NC = 2
NS = 16
CHUNK = 80


def _seg_sum_sc(n_nodes, n_edges, feat, with_cnt):
  nw = NC * NS
  epw = n_edges // nw
  iters = epw // CHUNK
  assert iters % 2 == 1 and iters * CHUNK * nw == n_edges
  rpt = (-(-n_nodes // NS) + 7) // 8 * 8
  last = n_nodes - (NS - 1) * rpt

  out_type = [jax.ShapeDtypeStruct((NC, n_nodes, feat), jnp.float32)]
  if with_cnt:
    out_type.append(jax.ShapeDtypeStruct((NC, n_nodes, 16), jnp.float32))

  scratch = [
      pltpu.VMEM((iters, CHUNK), jnp.int32),
      pltpu.VMEM((iters, CHUNK), jnp.int32),
      pltpu.VMEM((CHUNK, feat), jnp.float32),
      pltpu.VMEM((CHUNK, feat), jnp.float32),
      pltpu.VMEM_SHARED((n_nodes, feat), jnp.float32),
      pltpu.SemaphoreType.DMA,
  ]
  if with_cnt:
    scratch += [
        pltpu.VMEM((CHUNK, 16), jnp.float32),
        pltpu.VMEM_SHARED((n_nodes, 16), jnp.float32),
        pltpu.SemaphoreType.DMA,
    ]

  mesh = plsc.VectorSubcoreMesh(core_axis_name="c", subcore_axis_name="s")

  def body(*refs):
    if with_cnt:
      (p_hbm, srcm, dstm, z64, z16, ones_hbm, agg_out, cnt_out,
       src_v, dst_v, rows_a, rows_b, acc_sh, sem, ones_v, cnt_sh,
       sem_c) = refs
    else:
      (p_hbm, srcm, dstm, z64, agg_out,
       src_v, dst_v, rows_a, rows_b, acc_sh, sem) = refs
    c = lax.axis_index("c")
    s = lax.axis_index("s")
    wid = s * NC + c
    rbase = s * rpt

    def ranged(do):
      @pl.when(s < NS - 1)
      def _():
        do(rpt)
      @pl.when(s == NS - 1)
      def _():
        do(last)

    ranged(lambda sz: pltpu.sync_copy(z64.at[pl.ds(0, sz)],
                                      acc_sh.at[pl.ds(rbase, sz)]))
    if with_cnt:
      ranged(lambda sz: pltpu.sync_copy(z16.at[pl.ds(0, sz)],
                                        cnt_sh.at[pl.ds(rbase, sz)]))
      pltpu.sync_copy(ones_hbm, ones_v)
    pltpu.sync_copy(srcm.at[wid], src_v)
    pltpu.sync_copy(dstm.at[wid], dst_v)
    plsc.subcore_barrier()

    def consume(j, buf):
      pltpu.make_async_copy(p_hbm.at[src_v.at[j]], buf, sem).wait()
      if with_cnt:
        pltpu.async_copy(ones_v, cnt_sh.at[dst_v.at[j]], sem_c, add=True)
      pltpu.sync_copy(buf, acc_sh.at[dst_v.at[j]], add=True)

    pltpu.async_copy(p_hbm.at[src_v.at[0]], rows_a, sem)

    def pair(g, carry):
      j = 2 * g
      pltpu.async_copy(p_hbm.at[src_v.at[j + 1]], rows_b, sem)
      consume(j, rows_a)
      pltpu.async_copy(p_hbm.at[src_v.at[j + 2]], rows_a, sem)
      consume(j + 1, rows_b)
      return carry

    lax.fori_loop(0, (iters - 1) // 2, pair, 0)
    consume(iters - 1, rows_a)
    if with_cnt:
      def drain(j, carry):
        pltpu.make_async_copy(ones_v, cnt_sh.at[dst_v.at[j]], sem_c).wait()
        return carry
      lax.fori_loop(0, iters, drain, 0)
    plsc.subcore_barrier()
    ranged(lambda sz: pltpu.sync_copy(acc_sh.at[pl.ds(rbase, sz)],
                                      agg_out.at[c, pl.ds(rbase, sz)]))
    if with_cnt:
      ranged(lambda sz: pltpu.sync_copy(cnt_sh.at[pl.ds(rbase, sz)],
                                        cnt_out.at[c, pl.ds(rbase, sz)]))

  return pl.kernel(body, out_type=out_type, mesh=mesh, scratch_types=scratch,
                   compiler_params=pltpu.CompilerParams(
                       use_tc_tiling_on_sc=False))


def _tc1_body(x_ref, w_ref, p_ref, r_ref):
  y = jnp.dot(x_ref[...], w_ref[...], preferred_element_type=jnp.float32)
  dh = p_ref.shape[1]
  p_ref[...] = y[:, :dh]
  r_ref[...] = y[:, dh:]


def _tc2_body(n_nodes, a_ref, c_ref, r_ref, b_ref, w_ref, p_ref, r2_ref,
              hpre_s, cs_s):
  ph = pl.program_id(0)
  i = pl.program_id(1)
  blk = r_ref.shape[0]

  @pl.when(ph == 0)
  def _():
    agg = a_ref[0] + a_ref[1]
    cnt = c_ref[0, :, 0:1] + c_ref[1, :, 0:1]
    hpre = agg / jnp.maximum(cnt, 1.0) + b_ref[...] + r_ref[...]
    hpre_s[pl.ds(i * blk, blk), :] = hpre
    colsum = jnp.sum(hpre, axis=0, keepdims=True)
    @pl.when(i == 0)
    def _():
      cs_s[...] = colsum
    @pl.when(i > 0)
    def _():
      cs_s[...] = cs_s[...] + colsum

  @pl.when(ph == 1)
  def _():
    hpre = hpre_s[pl.ds(i * blk, blk), :]
    colmean = cs_s[...] * (1.0 / n_nodes)
    rnorm = jnp.sqrt(1e-6 + jnp.sum(hpre * hpre, axis=1, keepdims=True))
    h = jnp.maximum(hpre / rnorm - colmean, 0.0)
    y = jnp.dot(h, w_ref[...], preferred_element_type=jnp.float32)
    dh = p_ref.shape[1]
    p_ref[...] = y[:, :dh]
    r2_ref[...] = y[:, dh:]


def _tc3_body(a_ref, c_ref, r_ref, b_ref, o_ref):
  agg = a_ref[0] + a_ref[1]
  cnt = c_ref[0, :, 0:1] + c_ref[1, :, 0:1]
  o_ref[...] = agg / jnp.maximum(cnt, 1.0) + b_ref[...] + r_ref[...]


@jax.jit
def kernel(x, edge_index, Wl1, bl1, Wr1, Wl2, bl2, Wr2):
  n, din = x.shape
  e = edge_index.shape[1]
  dh = Wl1.shape[1]
  dout = Wl2.shape[1]

  nw = NC * NS
  srcm = edge_index[0].reshape(nw, e // (nw * CHUNK), CHUNK)
  dstm = edge_index[1].reshape(nw, e // (nw * CHUNK), CHUNK)
  rpt = (-(-n // NS) + 7) // 8 * 8
  z64 = jnp.zeros((rpt, dh), jnp.float32)
  z16 = jnp.zeros((rpt, 16), jnp.float32)
  ones16 = jnp.ones((CHUNK, 16), jnp.float32)
  wcat1 = jnp.concatenate([Wl1, Wr1], axis=1)
  wcat2 = jnp.concatenate([Wl2, Wr2], axis=1)

  nb = 10
  blk = n // nb

  p1, r1 = pl.pallas_call(
      _tc1_body,
      grid=(nb,),
      in_specs=[pl.BlockSpec((blk, din), lambda i: (i, 0)),
                pl.BlockSpec((din, 2 * dh), lambda i: (0, 0))],
      out_specs=[pl.BlockSpec((blk, dh), lambda i: (i, 0)),
                 pl.BlockSpec((blk, dh), lambda i: (i, 0))],
      out_shape=[jax.ShapeDtypeStruct((n, dh), jnp.float32),
                 jax.ShapeDtypeStruct((n, dh), jnp.float32)],
  )(x, wcat1)

  seg1 = _seg_sum_sc(n, e, dh, with_cnt=True)
  agg1, cnt16 = seg1(p1, srcm, dstm, z64, z16, ones16)

  p2, r2 = pl.pallas_call(
      functools.partial(_tc2_body, n),
      grid=(2, nb),
      in_specs=[pl.BlockSpec((2, blk, dh), lambda p, i: (0, i, 0)),
                pl.BlockSpec((2, blk, 16), lambda p, i: (0, i, 0)),
                pl.BlockSpec((blk, dh), lambda p, i: (i, 0)),
                pl.BlockSpec((1, dh), lambda p, i: (0, 0)),
                pl.BlockSpec((dh, 2 * dout), lambda p, i: (0, 0))],
      out_specs=[pl.BlockSpec((blk, dout), lambda p, i: (i, 0)),
                 pl.BlockSpec((blk, dout), lambda p, i: (i, 0))],
      out_shape=[jax.ShapeDtypeStruct((n, dout), jnp.float32),
                 jax.ShapeDtypeStruct((n, dout), jnp.float32)],
      scratch_shapes=[pltpu.VMEM((n, dh), jnp.float32),
                      pltpu.VMEM((1, dh), jnp.float32)],
  )(agg1, cnt16, r1, bl1.reshape(1, dh), wcat2)

  seg2 = _seg_sum_sc(n, e, dout, with_cnt=False)
  (agg2,) = seg2(p2, srcm, dstm, z64)

  out = pl.pallas_call(
      _tc3_body,
      grid=(nb,),
      in_specs=[pl.BlockSpec((2, blk, dout), lambda i: (0, i, 0)),
                pl.BlockSpec((2, blk, 16), lambda i: (0, i, 0)),
                pl.BlockSpec((blk, dout), lambda i: (i, 0)),
                pl.BlockSpec((1, dout), lambda i: (0, 0))],
      out_specs=pl.BlockSpec((blk, dout), lambda i: (i, 0)),
      out_shape=jax.ShapeDtypeStruct((n, dout), jnp.float32),
  )(agg2, cnt16, r2, bl2.reshape(1, dout))
  return out


# speedup vs baseline: 1.0341x; 1.0341x over previous
import functools

import jax
import jax.numpy as jnp
from jax import lax
from jax.experimental import pallas as pl
from jax.experimental.pallas import tpu as pltpu
from jax.experimental.pallas import tpu_sc as plsc

NC = 2
NS = 16
CHUNK = 80


def _seg_sum_sc(n_nodes, n_edges, feat, with_cnt):
  nw = NC * NS
  epw = n_edges // nw
  iters = epw // CHUNK
  assert iters % 2 == 1 and iters * CHUNK * nw == n_edges
  rpt = (-(-n_nodes // NS) + 7) // 8 * 8
  last = n_nodes - (NS - 1) * rpt

  out_type = [jax.ShapeDtypeStruct((NC, n_nodes, feat), jnp.float32)]
  if with_cnt:
    out_type.append(jax.ShapeDtypeStruct((NC, n_nodes, 16), jnp.float32))

  scratch = [
      pltpu.VMEM((iters, CHUNK), jnp.int32),
      pltpu.VMEM((iters, CHUNK), jnp.int32),
      pltpu.VMEM((CHUNK, feat), jnp.float32),
      pltpu.VMEM((CHUNK, feat), jnp.float32),
      pltpu.VMEM_SHARED((n_nodes, feat), jnp.float32),
      pltpu.SemaphoreType.DMA,
  ]
  if with_cnt:
    scratch += [
        pltpu.VMEM((CHUNK, 16), jnp.float32),
        pltpu.VMEM_SHARED((n_nodes, 16), jnp.float32),
        pltpu.SemaphoreType.DMA,
    ]

  mesh = plsc.VectorSubcoreMesh(core_axis_name="c", subcore_axis_name="s")

  def body(*refs):
    if with_cnt:
      (p_hbm, srcm, dstm, z64, z16, ones_hbm, agg_out, cnt_out,
       src_v, dst_v, rows_a, rows_b, acc_sh, sem, ones_v, cnt_sh,
       sem_c) = refs
    else:
      (p_hbm, srcm, dstm, z64, agg_out,
       src_v, dst_v, rows_a, rows_b, acc_sh, sem) = refs
    c = lax.axis_index("c")
    s = lax.axis_index("s")
    wid = s * NC + c
    rbase = s * rpt

    def ranged(do):
      @pl.when(s < NS - 1)
      def _():
        do(rpt)
      @pl.when(s == NS - 1)
      def _():
        do(last)

    ranged(lambda sz: pltpu.sync_copy(z64.at[pl.ds(0, sz)],
                                      acc_sh.at[pl.ds(rbase, sz)]))
    if with_cnt:
      ranged(lambda sz: pltpu.sync_copy(z16.at[pl.ds(0, sz)],
                                        cnt_sh.at[pl.ds(rbase, sz)]))
      pltpu.sync_copy(ones_hbm, ones_v)
    pltpu.sync_copy(srcm.at[wid], src_v)
    pltpu.sync_copy(dstm.at[wid], dst_v)
    plsc.subcore_barrier()

    def consume(j, buf):
      pltpu.make_async_copy(p_hbm.at[src_v.at[j]], buf, sem).wait()
      if with_cnt:
        pltpu.async_copy(ones_v, cnt_sh.at[dst_v.at[j]], sem_c, add=True)
      pltpu.sync_copy(buf, acc_sh.at[dst_v.at[j]], add=True)

    pltpu.async_copy(p_hbm.at[src_v.at[0]], rows_a, sem)

    def pair(g, carry):
      j = 2 * g
      pltpu.async_copy(p_hbm.at[src_v.at[j + 1]], rows_b, sem)
      consume(j, rows_a)
      pltpu.async_copy(p_hbm.at[src_v.at[j + 2]], rows_a, sem)
      consume(j + 1, rows_b)
      return carry

    lax.fori_loop(0, (iters - 1) // 2, pair, 0)
    consume(iters - 1, rows_a)
    if with_cnt:
      def drain(j, carry):
        pltpu.make_async_copy(ones_v, cnt_sh.at[dst_v.at[j]], sem_c).wait()
        return carry
      lax.fori_loop(0, iters, drain, 0)
    plsc.subcore_barrier()
    ranged(lambda sz: pltpu.sync_copy(acc_sh.at[pl.ds(rbase, sz)],
                                      agg_out.at[c, pl.ds(rbase, sz)]))
    if with_cnt:
      ranged(lambda sz: pltpu.sync_copy(cnt_sh.at[pl.ds(rbase, sz)],
                                        cnt_out.at[c, pl.ds(rbase, sz)]))

  return pl.kernel(body, out_type=out_type, mesh=mesh, scratch_types=scratch,
                   compiler_params=pltpu.CompilerParams(
                       use_tc_tiling_on_sc=False))


def _tc1_body(x_ref, w_ref, p_ref, r_ref):
  y = jnp.dot(x_ref[...], w_ref[...], preferred_element_type=jnp.float32)
  dh = p_ref.shape[1]
  p_ref[...] = y[:, :dh]
  r_ref[...] = y[:, dh:]


def _tc2_body(a_ref, c_ref, r_ref, b_ref, w_ref, p_ref, r2_ref):
  agg = a_ref[0] + a_ref[1]
  cnt = c_ref[0, :, 0:1] + c_ref[1, :, 0:1]
  hpre = agg / jnp.maximum(cnt, 1.0) + b_ref[...] + r_ref[...]
  colmean = jnp.mean(hpre, axis=0, keepdims=True)
  rnorm = jnp.sqrt(1e-6 + jnp.sum(hpre * hpre, axis=1, keepdims=True))
  h = jnp.maximum(hpre / rnorm - colmean, 0.0)
  y = jnp.dot(h, w_ref[...], preferred_element_type=jnp.float32)
  dh = p_ref.shape[1]
  p_ref[...] = y[:, :dh]
  r2_ref[...] = y[:, dh:]


def _tc3_body(a_ref, c_ref, r_ref, b_ref, o_ref):
  agg = a_ref[0] + a_ref[1]
  cnt = c_ref[0, :, 0:1] + c_ref[1, :, 0:1]
  o_ref[...] = agg / jnp.maximum(cnt, 1.0) + b_ref[...] + r_ref[...]


@jax.jit
def kernel(x, edge_index, Wl1, bl1, Wr1, Wl2, bl2, Wr2):
  n, din = x.shape
  e = edge_index.shape[1]
  dh = Wl1.shape[1]
  dout = Wl2.shape[1]

  nw = NC * NS
  srcm = edge_index[0].reshape(nw, e // (nw * CHUNK), CHUNK)
  dstm = edge_index[1].reshape(nw, e // (nw * CHUNK), CHUNK)
  rpt = (-(-n // NS) + 7) // 8 * 8
  z64 = jnp.zeros((rpt, dh), jnp.float32)
  z16 = jnp.zeros((rpt, 16), jnp.float32)
  ones16 = jnp.ones((CHUNK, 16), jnp.float32)
  wcat1 = jnp.concatenate([Wl1, Wr1], axis=1)
  wcat2 = jnp.concatenate([Wl2, Wr2], axis=1)

  nb = 10
  blk = n // nb

  p1, r1 = pl.pallas_call(
      _tc1_body,
      grid=(nb,),
      in_specs=[pl.BlockSpec((blk, din), lambda i: (i, 0)),
                pl.BlockSpec((din, 2 * dh), lambda i: (0, 0))],
      out_specs=[pl.BlockSpec((blk, dh), lambda i: (i, 0)),
                 pl.BlockSpec((blk, dh), lambda i: (i, 0))],
      out_shape=[jax.ShapeDtypeStruct((n, dh), jnp.float32),
                 jax.ShapeDtypeStruct((n, dh), jnp.float32)],
  )(x, wcat1)

  seg1 = _seg_sum_sc(n, e, dh, with_cnt=True)
  agg1, cnt16 = seg1(p1, srcm, dstm, z64, z16, ones16)

  p2, r2 = pl.pallas_call(
      _tc2_body,
      out_shape=[jax.ShapeDtypeStruct((n, dout), jnp.float32),
                 jax.ShapeDtypeStruct((n, dout), jnp.float32)],
  )(agg1, cnt16, r1, bl1.reshape(1, dh), wcat2)

  seg2 = _seg_sum_sc(n, e, dout, with_cnt=False)
  (agg2,) = seg2(p2, srcm, dstm, z64)

  out = pl.pallas_call(
      _tc3_body,
      grid=(nb,),
      in_specs=[pl.BlockSpec((2, blk, dout), lambda i: (0, i, 0)),
                pl.BlockSpec((2, blk, 16), lambda i: (0, i, 0)),
                pl.BlockSpec((blk, dout), lambda i: (i, 0)),
                pl.BlockSpec((1, dout), lambda i: (0, 0))],
      out_specs=pl.BlockSpec((blk, dout), lambda i: (i, 0)),
      out_shape=jax.ShapeDtypeStruct((n, dout), jnp.float32),
  )(agg2, cnt16, r2, bl2.reshape(1, dout))
  return out


# speedup vs baseline: 1.0560x; 1.0212x over previous
import jax
import jax.numpy as jnp
from jax import lax
from jax.experimental import pallas as pl
from jax.experimental.pallas import tpu as pltpu
from jax.experimental.pallas import tpu_sc as plsc

NC = 2
NS = 16
CHUNK = 80


def _seg_sum_sc(n_nodes, n_edges, feat, with_cnt):
  nw = NC * NS
  epw = n_edges // nw
  iters = epw // CHUNK
  assert iters % 2 == 1 and iters * CHUNK * nw == n_edges
  rpt = (-(-n_nodes // NS) + 7) // 8 * 8
  last = n_nodes - (NS - 1) * rpt

  out_type = [jax.ShapeDtypeStruct((NC, n_nodes, feat), jnp.float32)]
  if with_cnt:
    out_type.append(jax.ShapeDtypeStruct((NC, n_nodes, 16), jnp.float32))

  scratch = [
      pltpu.VMEM((iters, CHUNK), jnp.int32),
      pltpu.VMEM((iters, CHUNK), jnp.int32),
      pltpu.VMEM((CHUNK, feat), jnp.float32),
      pltpu.VMEM((CHUNK, feat), jnp.float32),
      pltpu.VMEM_SHARED((n_nodes, feat), jnp.float32),
      pltpu.SemaphoreType.DMA,
  ]
  if with_cnt:
    scratch += [
        pltpu.VMEM((CHUNK, 16), jnp.float32),
        pltpu.VMEM_SHARED((n_nodes, 16), jnp.float32),
        pltpu.SemaphoreType.DMA,
    ]

  mesh = plsc.VectorSubcoreMesh(core_axis_name="c", subcore_axis_name="s")

  def body(*refs):
    if with_cnt:
      (p_hbm, srcm, dstm, z64, z16, ones_hbm, agg_out, cnt_out,
       src_v, dst_v, rows_a, rows_b, acc_sh, sem, ones_v, cnt_sh,
       sem_c) = refs
    else:
      (p_hbm, srcm, dstm, z64, agg_out,
       src_v, dst_v, rows_a, rows_b, acc_sh, sem) = refs
    c = lax.axis_index("c")
    s = lax.axis_index("s")
    wid = s * NC + c
    rbase = s * rpt

    def ranged(do):
      @pl.when(s < NS - 1)
      def _():
        do(rpt)
      @pl.when(s == NS - 1)
      def _():
        do(last)

    ranged(lambda sz: pltpu.sync_copy(z64.at[pl.ds(0, sz)],
                                      acc_sh.at[pl.ds(rbase, sz)]))
    if with_cnt:
      ranged(lambda sz: pltpu.sync_copy(z16.at[pl.ds(0, sz)],
                                        cnt_sh.at[pl.ds(rbase, sz)]))
      pltpu.sync_copy(ones_hbm, ones_v)
    pltpu.sync_copy(srcm.at[wid], src_v)
    pltpu.sync_copy(dstm.at[wid], dst_v)
    plsc.subcore_barrier()

    def consume(j, buf):
      pltpu.make_async_copy(p_hbm.at[src_v.at[j]], buf, sem).wait()
      if with_cnt:
        pltpu.async_copy(ones_v, cnt_sh.at[dst_v.at[j]], sem_c, add=True)
      pltpu.sync_copy(buf, acc_sh.at[dst_v.at[j]], add=True)

    pltpu.async_copy(p_hbm.at[src_v.at[0]], rows_a, sem)

    def pair(g, carry):
      j = 2 * g
      pltpu.async_copy(p_hbm.at[src_v.at[j + 1]], rows_b, sem)
      consume(j, rows_a)
      pltpu.async_copy(p_hbm.at[src_v.at[j + 2]], rows_a, sem)
      consume(j + 1, rows_b)
      return carry

    lax.fori_loop(0, (iters - 1) // 2, pair, 0)
    consume(iters - 1, rows_a)
    if with_cnt:
      def drain(j, carry):
        pltpu.make_async_copy(ones_v, cnt_sh.at[dst_v.at[j]], sem_c).wait()
        return carry
      lax.fori_loop(0, iters, drain, 0)
    plsc.subcore_barrier()
    ranged(lambda sz: pltpu.sync_copy(acc_sh.at[pl.ds(rbase, sz)],
                                      agg_out.at[c, pl.ds(rbase, sz)]))
    if with_cnt:
      ranged(lambda sz: pltpu.sync_copy(cnt_sh.at[pl.ds(rbase, sz)],
                                        cnt_out.at[c, pl.ds(rbase, sz)]))

  return pl.kernel(body, out_type=out_type, mesh=mesh, scratch_types=scratch,
                   compiler_params=pltpu.CompilerParams(
                       use_tc_tiling_on_sc=False))


def _tc1_body(x_ref, w_ref, p_ref, r_ref):
  y = jnp.dot(x_ref[...], w_ref[...], preferred_element_type=jnp.float32)
  dh = p_ref.shape[1]
  p_ref[...] = y[:, :dh]
  r_ref[...] = y[:, dh:]


def _tc2_body(a_ref, c_ref, r_ref, b_ref, w_ref, p_ref, r2_ref):
  agg = a_ref[0] + a_ref[1]
  cnt = c_ref[0, :, 0:1] + c_ref[1, :, 0:1]
  hpre = agg / jnp.maximum(cnt, 1.0) + b_ref[...] + r_ref[...]
  colmean = jnp.mean(hpre, axis=0, keepdims=True)
  rnorm = jnp.sqrt(1e-6 + jnp.sum(hpre * hpre, axis=1, keepdims=True))
  h = jnp.maximum(hpre / rnorm - colmean, 0.0)
  y = jnp.dot(h, w_ref[...], preferred_element_type=jnp.float32)
  dh = p_ref.shape[1]
  p_ref[...] = y[:, :dh]
  r2_ref[...] = y[:, dh:]


def _tc3_body(a_ref, c_ref, r_ref, b_ref, o_ref):
  agg = a_ref[0] + a_ref[1]
  cnt = c_ref[0, :, 0:1] + c_ref[1, :, 0:1]
  o_ref[...] = agg / jnp.maximum(cnt, 1.0) + b_ref[...] + r_ref[...]


@jax.jit
def kernel(x, edge_index, Wl1, bl1, Wr1, Wl2, bl2, Wr2):
  n, din = x.shape
  e = edge_index.shape[1]
  dh = Wl1.shape[1]
  dout = Wl2.shape[1]

  nw = NC * NS
  srcm = edge_index[0].reshape(nw, e // (nw * CHUNK), CHUNK)
  dstm = edge_index[1].reshape(nw, e // (nw * CHUNK), CHUNK)
  rpt = (-(-n // NS) + 7) // 8 * 8
  z64 = jnp.zeros((rpt, dh), jnp.float32)
  z16 = jnp.zeros((rpt, 16), jnp.float32)
  ones16 = jnp.ones((CHUNK, 16), jnp.float32)
  wcat1 = jnp.concatenate([Wl1, Wr1], axis=1)
  wcat2 = jnp.concatenate([Wl2, Wr2], axis=1)

  p1, r1 = pl.pallas_call(
      _tc1_body,
      out_shape=[jax.ShapeDtypeStruct((n, dh), jnp.float32),
                 jax.ShapeDtypeStruct((n, dh), jnp.float32)],
  )(x, wcat1)

  seg1 = _seg_sum_sc(n, e, dh, with_cnt=True)
  agg1, cnt16 = seg1(p1, srcm, dstm, z64, z16, ones16)

  p2, r2 = pl.pallas_call(
      _tc2_body,
      out_shape=[jax.ShapeDtypeStruct((n, dout), jnp.float32),
                 jax.ShapeDtypeStruct((n, dout), jnp.float32)],
  )(agg1, cnt16, r1, bl1.reshape(1, dh), wcat2)

  seg2 = _seg_sum_sc(n, e, dout, with_cnt=False)
  (agg2,) = seg2(p2, srcm, dstm, z64)

  out = pl.pallas_call(
      _tc3_body,
      out_shape=jax.ShapeDtypeStruct((n, dout), jnp.float32),
  )(agg2, cnt16, r2, bl2.reshape(1, dout))
  return out
